# baseline (speedup 1.0000x reference)
import functools

import jax
import jax.numpy as jnp
from jax import lax
from jax.experimental import pallas as pl
from jax.experimental.pallas import tpu as pltpu
from jax.experimental.pallas import tpu_sc as plsc

T_OUT = 2048


@functools.cache
def _make_unpack(N, D, B):
    info = plsc.get_sparse_core_info()
    NC, NS, L = info.num_cores, info.num_subcores, info.num_lanes
    NW = NC * NS
    PW = (B * T_OUT) // NW
    assert PW * NW == B * T_OUT and T_OUT % PW == 0
    CH = T_OUT // PW
    G = 16
    NG = PW // G
    NBUF = 4

    mesh = plsc.VectorSubcoreMesh(core_axis_name="c", subcore_axis_name="s")

    @functools.partial(
        pl.kernel,
        mesh=mesh,
        out_type=jax.ShapeDtypeStruct((B * T_OUT, D), jnp.float32),
        scratch_types=[
            pltpu.VMEM((PW,), jnp.int32),
            pltpu.VMEM((L,), jnp.int32),
            *[pltpu.VMEM((G, D), jnp.float32) for _ in range(NBUF)],
            pltpu.VMEM((G, D), jnp.float32),
            *[pltpu.SemaphoreType.DMA for _ in range(2 * NBUF + 1)],
        ],
    )
    def unpack(data_hbm, len_hbm, out_hbm, idx_v, len_v, *rest):
        bufs = rest[:NBUF]
        zbuf = rest[NBUF]
        gsems = rest[NBUF + 1:2 * NBUF + 1]
        wsems = rest[2 * NBUF + 1:3 * NBUF + 1]
        zsem = rest[3 * NBUF + 1]
        wid = lax.axis_index("s") * NC + lax.axis_index("c")
        b = wid // CH
        t0 = (wid % CH) * PW
        row0 = wid * PW

        len_v[...] = jnp.zeros((L,), jnp.int32)
        pltpu.sync_copy(len_hbm, len_v.at[pl.ds(0, B)])
        lanes = lax.iota(jnp.int32, L)
        lv = len_v[...]
        lens = [lv[j] for j in range(B)]
        len_b = lens[0] * 0
        for j in range(B):
            len_b = jnp.where(b == j, lens[j], len_b)
        v = jnp.clip(len_b - t0, 0, PW)

        def idx_fill(s):
            t_vec = t0 + s * L + lanes
            acc = jnp.zeros((L,), jnp.int32)
            for lj in lens:
                acc = acc + jnp.minimum(t_vec, lj)
            idx_v[pl.ds(s * L, L)] = jnp.minimum(acc + b, N - 1)

        def gather(g, p):
            return pltpu.make_async_copy(
                data_hbm.at[idx_v.at[pl.ds(g * G, G)]], bufs[p], gsems[p]
            )

        def write(g, p):
            return pltpu.make_async_copy(
                bufs[p], out_hbm.at[pl.ds(row0 + g * G, G)], wsems[p]
            )

        NPRO = min(NBUF, NG)
        for s in range((NPRO * G + L - 1) // L):
            idx_fill(s)
        for g in range(NPRO):
            @pl.when(g * G < v)
            def _(g=g):
                gather(g, g % NBUF).start()

        def zrow(i, carry):
            for c in range(D // L):
                zbuf[i, pl.ds(c * L, L)] = jnp.zeros((L,), jnp.float32)
            return carry

        lax.fori_loop(0, G, zrow, 0)

        for g in range(NG):
            @pl.when(g * G >= v)
            def _(g=g):
                pltpu.make_async_copy(
                    zbuf, out_hbm.at[pl.ds(row0 + g * G, G)], zsem
                ).start()

        for s in range((NPRO * G + L - 1) // L, PW // L):
            idx_fill(s)

        for g in range(NG):
            p = g % NBUF

            @pl.when(g * G < v)
            def _(g=g, p=p):
                gather(g, p).wait()

                @pl.when(v < (g + 1) * G)
                def _():
                    def zfix(i, carry):
                        for c in range(D // L):
                            bufs[p][i, pl.ds(c * L, L)] = jnp.zeros(
                                (L,), jnp.float32)
                        return carry

                    lax.fori_loop(v - g * G, G, zfix, 0)

                write(g, p).start()

            if g + NBUF < NG:
                @pl.when((g + NBUF) * G < v)
                def _(g=g, p=p):
                    write(g, p).wait()
                    gather(g + NBUF, p).start()

        for g in range(NG):
            p = g % NBUF
            in_loop = (g + NBUF) * G < v if g + NBUF < NG else False

            @pl.when((g * G < v) & jnp.logical_not(in_loop))
            def _(g=g, p=p):
                write(g, p).wait()

            @pl.when(g * G >= v)
            def _(g=g):
                pltpu.make_async_copy(
                    zbuf, out_hbm.at[pl.ds(row0 + g * G, G)], zsem
                ).wait()

    return unpack


def kernel(data, lengths):
    N, D = data.shape
    B = lengths.shape[0]
    out = _make_unpack(N, D, B)(data, lengths.astype(jnp.int32))
    return out.reshape(B, T_OUT, D), lengths


# speedup vs baseline: 1.0577x; 1.0577x over previous
import functools

import jax
import jax.numpy as jnp
from jax import lax
from jax.experimental import pallas as pl
from jax.experimental.pallas import tpu as pltpu
from jax.experimental.pallas import tpu_sc as plsc

T_OUT = 2048


@functools.cache
def _make_unpack(N, D, B):
    info = plsc.get_sparse_core_info()
    NC, NS, L = info.num_cores, info.num_subcores, info.num_lanes
    NW = NC * NS
    PW = (B * T_OUT) // NW
    assert PW * NW == B * T_OUT and T_OUT % PW == 0
    CH = T_OUT // PW
    G = 32
    NG = PW // G
    NBUF = 3
    ZR = 16

    mesh = plsc.VectorSubcoreMesh(core_axis_name="c", subcore_axis_name="s")

    @functools.partial(
        pl.kernel,
        mesh=mesh,
        out_type=jax.ShapeDtypeStruct((B * T_OUT, D), jnp.float32),
        scratch_types=[
            pltpu.VMEM((PW,), jnp.int32),
            pltpu.VMEM((L,), jnp.int32),
            *[pltpu.VMEM((G, D), jnp.float32) for _ in range(NBUF)],
            pltpu.VMEM((ZR, D), jnp.float32),
            *[pltpu.SemaphoreType.DMA for _ in range(2 * NBUF + 1)],
        ],
    )
    def unpack(data_hbm, len_hbm, out_hbm, idx_v, len_v, *rest):
        bufs = rest[:NBUF]
        zbuf = rest[NBUF]
        gsems = rest[NBUF + 1:2 * NBUF + 1]
        wsems = rest[2 * NBUF + 1:3 * NBUF + 1]
        zsem = rest[3 * NBUF + 1]
        wid = lax.axis_index("s") * NC + lax.axis_index("c")
        b = wid // CH
        t0 = (wid % CH) * PW
        row0 = wid * PW

        len_v[...] = jnp.zeros((L,), jnp.int32)
        pltpu.sync_copy(len_hbm, len_v.at[pl.ds(0, B)])
        lanes = lax.iota(jnp.int32, L)
        lv = len_v[...]
        lens = [lv[j] for j in range(B)]
        len_b = lens[0] * 0
        for j in range(B):
            len_b = jnp.where(b == j, lens[j], len_b)
        v = jnp.clip(len_b - t0, 0, PW)

        def idx_fill(s):
            t_vec = t0 + s * L + lanes
            acc = jnp.zeros((L,), jnp.int32)
            for lj in lens:
                acc = acc + jnp.minimum(t_vec, lj)
            idx_v[pl.ds(s * L, L)] = jnp.minimum(acc + b, N - 1)

        def gather(g, p):
            return pltpu.make_async_copy(
                data_hbm.at[idx_v.at[pl.ds(g * G, G)]], bufs[p], gsems[p]
            )

        def write(g, p):
            return pltpu.make_async_copy(
                bufs[p], out_hbm.at[pl.ds(row0 + g * G, G)], wsems[p]
            )

        NPRO = min(NBUF, NG)
        for s in range((NPRO * G + L - 1) // L):
            idx_fill(s)
        for g in range(NPRO):
            @pl.when(g * G < v)
            def _(g=g):
                gather(g, g % NBUF).start()

        def zrow(i, carry):
            for c in range(D // L):
                zbuf[i, pl.ds(c * L, L)] = jnp.zeros((L,), jnp.float32)
            return carry

        lax.fori_loop(0, ZR, zrow, 0)

        for g in range(NG):
            @pl.when(g * G >= v)
            def _(g=g):
                for q in range(G // ZR):
                    pltpu.make_async_copy(
                        zbuf,
                        out_hbm.at[pl.ds(row0 + g * G + q * ZR, ZR)],
                        zsem,
                    ).start()

        for s in range((NPRO * G + L - 1) // L, PW // L):
            idx_fill(s)

        for g in range(NG):
            p = g % NBUF

            @pl.when(g * G < v)
            def _(g=g, p=p):
                gather(g, p).wait()

                @pl.when(v < (g + 1) * G)
                def _():
                    def zfix(i, carry):
                        for c in range(D // L):
                            bufs[p][i, pl.ds(c * L, L)] = jnp.zeros(
                                (L,), jnp.float32)
                        return carry

                    lax.fori_loop(v - g * G, G, zfix, 0)

                write(g, p).start()

            if g + NBUF < NG:
                @pl.when((g + NBUF) * G < v)
                def _(g=g, p=p):
                    write(g, p).wait()
                    gather(g + NBUF, p).start()

        for g in range(NG):
            p = g % NBUF
            in_loop = (g + NBUF) * G < v if g + NBUF < NG else False

            @pl.when((g * G < v) & jnp.logical_not(in_loop))
            def _(g=g, p=p):
                write(g, p).wait()

            @pl.when(g * G >= v)
            def _(g=g):
                for q in range(G // ZR):
                    pltpu.make_async_copy(
                        zbuf,
                        out_hbm.at[pl.ds(row0 + g * G + q * ZR, ZR)],
                        zsem,
                    ).wait()

    return unpack


def kernel(data, lengths):
    N, D = data.shape
    B = lengths.shape[0]
    out = _make_unpack(N, D, B)(data, lengths.astype(jnp.int32))
    return out.reshape(B, T_OUT, D), lengths
